# baseline (speedup 1.0000x reference)
import functools

import jax
import jax.numpy as jnp
from jax import lax
from jax.experimental import pallas as pl
from jax.experimental.pallas import tpu as pltpu
from jax.experimental.pallas import tpu_sc as plsc

N = 10000
E = 320000
G = 64
D = 128

BM = 1000
NBLK = N // BM

NC = 2
NS = 16
NW = NC * NS
EPW = E // NW
CH = 80
NCHUNK = EPW // CH
NROWCH = N // CH
ROWIT = (NROWCH + NS - 1) // NS
RB = 4
XB = 8
NSLOT = NCHUNK + 3
NOUT = (NSLOT + XB - 1) // XB


def _mm_body(x_ref, w_ref, o_ref):
    o_ref[...] = jnp.dot(x_ref[...], w_ref[...],
                         preferred_element_type=jnp.float32)


def _mm(x, w):
    return pl.pallas_call(
        _mm_body,
        grid=(NBLK,),
        in_specs=[pl.BlockSpec((BM, D), lambda i: (i, 0)),
                  pl.BlockSpec((D, D), lambda i: (0, 0))],
        out_specs=pl.BlockSpec((BM, D), lambda i: (i, 0)),
        out_shape=jax.ShapeDtypeStruct((N, D), jnp.float32),
    )(x, w)


def _gru_body(a_ref, b_ref, h_ref, wiT_ref, whT_ref, bi_ref, bh_ref, o_ref,
              *, leaky):
    agg = a_ref[...] + b_ref[...]
    h = h_ref[...]
    gi = jnp.dot(agg, wiT_ref[...], preferred_element_type=jnp.float32) \
        + bi_ref[...]
    gh = jnp.dot(h, whT_ref[...], preferred_element_type=jnp.float32) \
        + bh_ref[...]
    i_r, i_z, i_n = gi[:, 0:D], gi[:, D:2 * D], gi[:, 2 * D:3 * D]
    h_r, h_z, h_n = gh[:, 0:D], gh[:, D:2 * D], gh[:, 2 * D:3 * D]
    r = jax.nn.sigmoid(i_r + h_r)
    z = jax.nn.sigmoid(i_z + h_z)
    n = jnp.tanh(i_n + r * h_n)
    o = (1.0 - z) * n + z * h
    if leaky:
        o = jnp.where(o >= 0, o, 0.01 * o)
    o_ref[...] = o


def _gru(a, b, h, wiT, whT, bi, bh, leaky):
    return pl.pallas_call(
        functools.partial(_gru_body, leaky=leaky),
        grid=(NBLK,),
        in_specs=[pl.BlockSpec((BM, D), lambda i: (i, 0)),
                  pl.BlockSpec((BM, D), lambda i: (i, 0)),
                  pl.BlockSpec((BM, D), lambda i: (i, 0)),
                  pl.BlockSpec((D, 3 * D), lambda i: (0, 0)),
                  pl.BlockSpec((D, 3 * D), lambda i: (0, 0)),
                  pl.BlockSpec((1, 3 * D), lambda i: (0, 0)),
                  pl.BlockSpec((1, 3 * D), lambda i: (0, 0))],
        out_specs=pl.BlockSpec((BM, D), lambda i: (i, 0)),
        out_shape=jax.ShapeDtypeStruct((N, D), jnp.float32),
    )(a, b, h, wiT, whT, bi, bh)


def _stats1_body(x_ref, s1_ref):
    i = pl.program_id(0)
    ps1 = jnp.sum(x_ref[...], axis=0, keepdims=True)

    @pl.when(i == 0)
    def _():
        s1_ref[...] = ps1

    @pl.when(i > 0)
    def _():
        s1_ref[...] = s1_ref[...] + ps1


def _stats1(x):
    return pl.pallas_call(
        _stats1_body,
        grid=(NBLK,),
        in_specs=[pl.BlockSpec((BM, D), lambda i: (i, 0))],
        out_specs=pl.BlockSpec((1, D), lambda i: (0, 0)),
        out_shape=jax.ShapeDtypeStruct((1, D), jnp.float32),
    )(x)


def _stats2_body(x_ref, s1_ref, ms_ref, s2_ref):
    i = pl.program_id(0)
    cm = s1_ref[...] * (1.0 / N) * ms_ref[...]
    d = x_ref[...] - cm
    ps2 = jnp.sum(d * d, axis=0, keepdims=True)

    @pl.when(i == 0)
    def _():
        s2_ref[...] = ps2

    @pl.when(i > 0)
    def _():
        s2_ref[...] = s2_ref[...] + ps2


def _stats2(x, s1, ms):
    return pl.pallas_call(
        _stats2_body,
        grid=(NBLK,),
        in_specs=[pl.BlockSpec((BM, D), lambda i: (i, 0)),
                  pl.BlockSpec((1, D), lambda i: (0, 0)),
                  pl.BlockSpec((1, D), lambda i: (0, 0))],
        out_specs=pl.BlockSpec((1, D), lambda i: (0, 0)),
        out_shape=jax.ShapeDtypeStruct((1, D), jnp.float32),
    )(x, s1, ms)


def _norm_body(x_ref, s1_ref, s2_ref, w_ref, b_ref, ms_ref, o_ref, *,
               leaky_after):
    cm = s1_ref[...] * (1.0 / N) * ms_ref[...]
    var = s2_ref[...] * (1.0 / N)
    rstd = jax.lax.rsqrt(var + 1e-5)
    y = w_ref[...] * (x_ref[...] - cm) * rstd + b_ref[...]
    if leaky_after:
        y = jnp.where(y >= 0, y, 0.01 * y)
    o_ref[...] = y


def _norm(x, s1, s2, w, b, ms, leaky_after):
    return pl.pallas_call(
        functools.partial(_norm_body, leaky_after=leaky_after),
        grid=(NBLK,),
        in_specs=[pl.BlockSpec((BM, D), lambda i: (i, 0)),
                  pl.BlockSpec((1, D), lambda i: (0, 0)),
                  pl.BlockSpec((1, D), lambda i: (0, 0)),
                  pl.BlockSpec((1, D), lambda i: (0, 0)),
                  pl.BlockSpec((1, D), lambda i: (0, 0)),
                  pl.BlockSpec((1, D), lambda i: (0, 0))],
        out_specs=pl.BlockSpec((BM, D), lambda i: (i, 0)),
        out_shape=jax.ShapeDtypeStruct((N, D), jnp.float32),
    )(x, s1, s2, w, b, ms)


def _pool_body(b_ref, x_ref, o_ref):
    i = pl.program_id(0)
    b = b_ref[0, 0, :]
    rows = lax.broadcasted_iota(jnp.int32, (G, BM), 0)
    oh = (rows == b[None, :]).astype(jnp.float32)
    acc = jnp.dot(oh, x_ref[...], preferred_element_type=jnp.float32)

    @pl.when(i == 0)
    def _():
        o_ref[...] = acc

    @pl.when(i > 0)
    def _():
        o_ref[...] = o_ref[...] + acc


def _pool(batch3d, x):
    return pl.pallas_call(
        _pool_body,
        grid=(NBLK,),
        in_specs=[pl.BlockSpec((1, 1, BM), lambda i: (i, 0, 0)),
                  pl.BlockSpec((BM, D), lambda i: (i, 0))],
        out_specs=pl.BlockSpec((G, D), lambda i: (0, 0)),
        out_shape=jax.ShapeDtypeStruct((G, D), jnp.float32),
    )(batch3d, x)


def _sc_agg_body(weighted, *refs):
    if weighted:
        (m_hbm, src_hbm, dst_hbm, ew_hbm, out_hbm,
         rows, srcb, dstb, ewb, acc_sh,
         gsems, ssems, xsems) = refs
    else:
        (m_hbm, src_hbm, dst_hbm, out_hbm,
         rows, srcb, dstb, acc_sh,
         gsems, ssems, xsems) = refs
        ew_hbm = ewb = None
    c = lax.axis_index("c")
    s = lax.axis_index("s")
    wid = s * NC + c
    ebase = wid * EPW

    def _idx_loads(k, slot):
        pltpu.async_copy(src_hbm.at[pl.ds(ebase + k * CH, CH)], srcb[slot],
                         xsems[slot])
        pltpu.async_copy(dst_hbm.at[pl.ds(ebase + k * CH, CH)], dstb[slot],
                         xsems[slot])
        if weighted:
            pltpu.async_copy(ew_hbm.at[pl.ds(ebase + k * CH, CH)], ewb[slot],
                             xsems[slot])

    def _idx_drain(slot):
        pltpu.make_async_copy(src_hbm.at[pl.ds(0, CH)], srcb[slot],
                              xsems[slot]).wait()
        pltpu.make_async_copy(dst_hbm.at[pl.ds(0, CH)], dstb[slot],
                              xsems[slot]).wait()
        if weighted:
            pltpu.make_async_copy(ew_hbm.at[pl.ds(0, CH)], ewb[slot],
                                  xsems[slot]).wait()

    def _zrow(i, _):
        for cc in range(D // 16):
            rows[0][i, pl.ds(cc * 16, 16)] = jnp.zeros((16,), jnp.float32)
        return 0
    lax.fori_loop(0, CH, _zrow, 0)
    for j in range(ROWIT):
        cid = j * NS + s
        @pl.when(cid < NROWCH)
        def _():
            pltpu.sync_copy(rows[0], acc_sh.at[pl.ds(cid * CH, CH)])
    plsc.subcore_barrier()

    for j in range(3):
        _idx_loads(j, j)
    for j in range(2):
        _idx_drain(j)
        pltpu.async_copy(m_hbm.at[srcb[j]], rows[j], gsems[j])

    def _outer(k2, _):
        for b8 in range(XB):
            k = k2 * XB + b8
            b4 = b8 % RB

            @pl.when((k >= 2) & (k < NCHUNK + 2))
            def _():
                pltpu.make_async_copy(rows[(b8 + 2) % RB],
                                      acc_sh.at[pl.ds(0, CH)],
                                      ssems[(b8 + 2) % RB]).wait()

            @pl.when(k + 2 < NCHUNK)
            def _():
                _idx_drain((b8 + 2) % XB)
                pltpu.async_copy(m_hbm.at[srcb[(b8 + 2) % XB]],
                                 rows[(b8 + 2) % RB], gsems[(b8 + 2) % RB])

            @pl.when(k < NCHUNK)
            def _():
                pltpu.make_async_copy(m_hbm.at[pl.ds(0, CH)], rows[b4],
                                      gsems[b4]).wait()
                if weighted:
                    def _scale(g, _):
                        wv = ewb[b8][pl.ds(g * 16, 16)]
                        for l in range(16):
                            wl = wv.at[jnp.full((16,), l, jnp.int32)].get(
                                mode="promise_in_bounds")
                            r = g * 16 + l
                            for cc in range(D // 16):
                                sl = pl.ds(cc * 16, 16)
                                rows[b4][r, sl] = rows[b4][r, sl] * wl
                        return 0
                    lax.fori_loop(0, CH // 16, _scale, 0)
                pltpu.async_copy(rows[b4], acc_sh.at[dstb[b8]], ssems[b4],
                                 add=True)

            @pl.when(k + 3 < NCHUNK)
            def _():
                _idx_loads(k + 3, (b8 + 3) % XB)
        return 0
    lax.fori_loop(0, NOUT, _outer, 0)
    plsc.subcore_barrier()

    for j in range(ROWIT):
        cid = j * NS + s
        @pl.when(cid < NROWCH)
        def _():
            pltpu.sync_copy(acc_sh.at[pl.ds(cid * CH, CH)], rows[j % RB])
            pltpu.sync_copy(rows[j % RB], out_hbm.at[c].at[pl.ds(cid * CH, CH)])


def _sc_agg(m, src, dst, ew):
    weighted = ew is not None
    mesh = plsc.VectorSubcoreMesh(core_axis_name="c", subcore_axis_name="s",
                                  num_cores=NC, num_subcores=NS)
    scratch = [
        [pltpu.VMEM((CH, D), jnp.float32) for _ in range(RB)],
        [pltpu.VMEM((CH,), jnp.int32) for _ in range(XB)],
        [pltpu.VMEM((CH,), jnp.int32) for _ in range(XB)],
    ]
    if weighted:
        scratch.append([pltpu.VMEM((CH,), jnp.float32) for _ in range(XB)])
    scratch += [
        pltpu.VMEM_SHARED((N, D), jnp.float32),
        [pltpu.SemaphoreType.DMA for _ in range(RB)],
        [pltpu.SemaphoreType.DMA for _ in range(RB)],
        [pltpu.SemaphoreType.DMA for _ in range(XB)],
    ]
    fn = pl.kernel(
        functools.partial(_sc_agg_body, weighted),
        out_type=jax.ShapeDtypeStruct((NC, N, D), jnp.float32),
        mesh=mesh,
        scratch_types=scratch,
    )
    if weighted:
        return fn(m, src, dst, ew)
    return fn(m, src, dst)


def _conv(o, W, wiT, whT, bi, bh, src, dst, ew, leaky_last):
    num_layers = W.shape[0]
    for i in range(num_layers):
        m = _mm(o, W[i])
        parts = _sc_agg(m, src, dst, ew)
        o = _gru(parts[0], parts[1], o, wiT, whT, bi, bh,
                 leaky=(leaky_last and i == num_layers - 1))
    return o


def kernel(x, edge_index, edge_weights, batch,
           c1W, c1wi, c1wh, c1bi, c1bh,
           c2W, c2wi, c2wh, c2bi, c2bh,
           c3W, c3wi, c3wh, c3bi, c3bh,
           c4W, c4wi, c4wh, c4bi, c4bh,
           c5W, c5wi, c5wh, c5bi, c5bh,
           g1w, g1b, g1m, g2w, g2b, g2m,
           g3w, g3b, g3m, g4w, g4b, g4m):
    src = edge_index[0]
    dst = edge_index[1]
    ew = edge_weights
    batch3d = batch.reshape(NBLK, 1, BM)

    def prep(wi, wh, bi, bh):
        return wi.T, wh.T, bi.reshape(1, 3 * D), bh.reshape(1, 3 * D)

    def gprep(w, b, ms):
        return w.reshape(1, D), b.reshape(1, D), ms.reshape(1, D)

    def normed(o, gw, gb, gm, leaky_after):
        wr, br, msr = gprep(gw, gb, gm)
        s1 = _stats1(o)
        s2 = _stats2(o, s1, msr)
        return _norm(o, s1, s2, wr, br, msr, leaky_after=leaky_after)

    o = _conv(x, c1W, *prep(c1wi, c1wh, c1bi, c1bh), src, dst, ew,
              leaky_last=False)
    o = normed(o, g1w, g1b, g1m, leaky_after=True)

    o = _conv(o, c2W, *prep(c2wi, c2wh, c2bi, c2bh), src, dst, ew,
              leaky_last=True)
    o = normed(o, g2w, g2b, g2m, leaky_after=False)

    o = _conv(o, c3W, *prep(c3wi, c3wh, c3bi, c3bh), src, dst, ew,
              leaky_last=True)
    o = normed(o, g3w, g3b, g3m, leaky_after=False)

    o = _conv(o, c4W, *prep(c4wi, c4wh, c4bi, c4bh), src, dst, ew,
              leaky_last=True)
    o = normed(o, g4w, g4b, g4m, leaky_after=False)

    o = _conv(o, c5W, *prep(c5wi, c5wh, c5bi, c5bh), src, dst, None,
              leaky_last=True)

    return _pool(batch3d, o)


# speedup vs baseline: 1.1042x; 1.1042x over previous
import functools

import jax
import jax.numpy as jnp
from jax import lax
from jax.experimental import pallas as pl
from jax.experimental.pallas import tpu as pltpu
from jax.experimental.pallas import tpu_sc as plsc

N = 10000
E = 320000
G = 64
D = 128

BM = 1000
NBLK = N // BM

NC = 2
NS = 16
NW = NC * NS
EPW = E // NW
CH = 80
NCHUNK = EPW // CH
NROWCH = N // CH
ROWIT = (NROWCH + NS - 1) // NS
NB = 3
NOUT = (NCHUNK + NB) // NB


def _mm_body(x_ref, w_ref, o_ref):
    o_ref[...] = jnp.dot(x_ref[...], w_ref[...],
                         preferred_element_type=jnp.float32)


def _mm(x, w):
    return pl.pallas_call(
        _mm_body,
        grid=(NBLK,),
        in_specs=[pl.BlockSpec((BM, D), lambda i: (i, 0)),
                  pl.BlockSpec((D, D), lambda i: (0, 0))],
        out_specs=pl.BlockSpec((BM, D), lambda i: (i, 0)),
        out_shape=jax.ShapeDtypeStruct((N, D), jnp.float32),
    )(x, w)


def _gru_body(a_ref, b_ref, h_ref, wiT_ref, whT_ref, bi_ref, bh_ref, o_ref,
              *, leaky):
    agg = a_ref[...] + b_ref[...]
    h = h_ref[...]
    gi = jnp.dot(agg, wiT_ref[...], preferred_element_type=jnp.float32) \
        + bi_ref[...]
    gh = jnp.dot(h, whT_ref[...], preferred_element_type=jnp.float32) \
        + bh_ref[...]
    i_r, i_z, i_n = gi[:, 0:D], gi[:, D:2 * D], gi[:, 2 * D:3 * D]
    h_r, h_z, h_n = gh[:, 0:D], gh[:, D:2 * D], gh[:, 2 * D:3 * D]
    r = jax.nn.sigmoid(i_r + h_r)
    z = jax.nn.sigmoid(i_z + h_z)
    n = jnp.tanh(i_n + r * h_n)
    o = (1.0 - z) * n + z * h
    if leaky:
        o = jnp.where(o >= 0, o, 0.01 * o)
    o_ref[...] = o


def _gru(a, b, h, wiT, whT, bi, bh, leaky):
    return pl.pallas_call(
        functools.partial(_gru_body, leaky=leaky),
        grid=(NBLK,),
        in_specs=[pl.BlockSpec((BM, D), lambda i: (i, 0)),
                  pl.BlockSpec((BM, D), lambda i: (i, 0)),
                  pl.BlockSpec((BM, D), lambda i: (i, 0)),
                  pl.BlockSpec((D, 3 * D), lambda i: (0, 0)),
                  pl.BlockSpec((D, 3 * D), lambda i: (0, 0)),
                  pl.BlockSpec((1, 3 * D), lambda i: (0, 0)),
                  pl.BlockSpec((1, 3 * D), lambda i: (0, 0))],
        out_specs=pl.BlockSpec((BM, D), lambda i: (i, 0)),
        out_shape=jax.ShapeDtypeStruct((N, D), jnp.float32),
    )(a, b, h, wiT, whT, bi, bh)


def _stats1_body(x_ref, s1_ref):
    i = pl.program_id(0)
    ps1 = jnp.sum(x_ref[...], axis=0, keepdims=True)

    @pl.when(i == 0)
    def _():
        s1_ref[...] = ps1

    @pl.when(i > 0)
    def _():
        s1_ref[...] = s1_ref[...] + ps1


def _stats1(x):
    return pl.pallas_call(
        _stats1_body,
        grid=(NBLK,),
        in_specs=[pl.BlockSpec((BM, D), lambda i: (i, 0))],
        out_specs=pl.BlockSpec((1, D), lambda i: (0, 0)),
        out_shape=jax.ShapeDtypeStruct((1, D), jnp.float32),
    )(x)


def _stats2_body(x_ref, s1_ref, ms_ref, s2_ref):
    i = pl.program_id(0)
    cm = s1_ref[...] * (1.0 / N) * ms_ref[...]
    d = x_ref[...] - cm
    ps2 = jnp.sum(d * d, axis=0, keepdims=True)

    @pl.when(i == 0)
    def _():
        s2_ref[...] = ps2

    @pl.when(i > 0)
    def _():
        s2_ref[...] = s2_ref[...] + ps2


def _stats2(x, s1, ms):
    return pl.pallas_call(
        _stats2_body,
        grid=(NBLK,),
        in_specs=[pl.BlockSpec((BM, D), lambda i: (i, 0)),
                  pl.BlockSpec((1, D), lambda i: (0, 0)),
                  pl.BlockSpec((1, D), lambda i: (0, 0))],
        out_specs=pl.BlockSpec((1, D), lambda i: (0, 0)),
        out_shape=jax.ShapeDtypeStruct((1, D), jnp.float32),
    )(x, s1, ms)


def _norm_body(x_ref, s1_ref, s2_ref, w_ref, b_ref, ms_ref, o_ref, *,
               leaky_after):
    cm = s1_ref[...] * (1.0 / N) * ms_ref[...]
    var = s2_ref[...] * (1.0 / N)
    rstd = jax.lax.rsqrt(var + 1e-5)
    y = w_ref[...] * (x_ref[...] - cm) * rstd + b_ref[...]
    if leaky_after:
        y = jnp.where(y >= 0, y, 0.01 * y)
    o_ref[...] = y


def _norm(x, s1, s2, w, b, ms, leaky_after):
    return pl.pallas_call(
        functools.partial(_norm_body, leaky_after=leaky_after),
        grid=(NBLK,),
        in_specs=[pl.BlockSpec((BM, D), lambda i: (i, 0)),
                  pl.BlockSpec((1, D), lambda i: (0, 0)),
                  pl.BlockSpec((1, D), lambda i: (0, 0)),
                  pl.BlockSpec((1, D), lambda i: (0, 0)),
                  pl.BlockSpec((1, D), lambda i: (0, 0)),
                  pl.BlockSpec((1, D), lambda i: (0, 0))],
        out_specs=pl.BlockSpec((BM, D), lambda i: (i, 0)),
        out_shape=jax.ShapeDtypeStruct((N, D), jnp.float32),
    )(x, s1, s2, w, b, ms)


def _pool_body(b_ref, x_ref, o_ref):
    i = pl.program_id(0)
    b = b_ref[0, 0, :]
    rows = lax.broadcasted_iota(jnp.int32, (G, BM), 0)
    oh = (rows == b[None, :]).astype(jnp.float32)
    acc = jnp.dot(oh, x_ref[...], preferred_element_type=jnp.float32)

    @pl.when(i == 0)
    def _():
        o_ref[...] = acc

    @pl.when(i > 0)
    def _():
        o_ref[...] = o_ref[...] + acc


def _pool(batch3d, x):
    return pl.pallas_call(
        _pool_body,
        grid=(NBLK,),
        in_specs=[pl.BlockSpec((1, 1, BM), lambda i: (i, 0, 0)),
                  pl.BlockSpec((BM, D), lambda i: (i, 0))],
        out_specs=pl.BlockSpec((G, D), lambda i: (0, 0)),
        out_shape=jax.ShapeDtypeStruct((G, D), jnp.float32),
    )(batch3d, x)


def _sc_agg_body(weighted, *refs):
    if weighted:
        (m_hbm, src_hbm, dst_hbm, ew_hbm, out_hbm,
         src_v, rows, dstb, ewb, acc_sh, gsems, ssems, dsems, esems) = refs
    else:
        (m_hbm, src_hbm, dst_hbm, out_hbm,
         src_v, rows, dstb, acc_sh, gsems, ssems, dsems) = refs
        ew_hbm = ewb = esems = None
    c = lax.axis_index("c")
    s = lax.axis_index("s")
    wid = s * NC + c

    pltpu.sync_copy(src_hbm.at[pl.ds(wid * EPW, EPW)], src_v)

    def _zrow(i, _):
        for cc in range(D // 16):
            rows[0][i, pl.ds(cc * 16, 16)] = jnp.zeros((16,), jnp.float32)
        return 0
    lax.fori_loop(0, CH, _zrow, 0)
    for j in range(ROWIT):
        cid = j * NS + s
        @pl.when(cid < NROWCH)
        def _():
            pltpu.sync_copy(rows[0], acc_sh.at[pl.ds(cid * CH, CH)])
    plsc.subcore_barrier()

    for j in range(NB - 1):
        pltpu.async_copy(m_hbm.at[src_v.at[pl.ds(j * CH, CH)]], rows[j],
                         gsems[j])
        pltpu.async_copy(dst_hbm.at[pl.ds(wid * EPW + j * CH, CH)], dstb[j],
                         dsems[j])
        if weighted:
            pltpu.async_copy(ew_hbm.at[pl.ds(wid * EPW + j * CH, CH)],
                             ewb[j], esems[j])

    def _outer(k2, _):
        for b in range(NB):
            k = k2 * NB + b
            bprev = (b + NB - 1) % NB

            @pl.when(k < NCHUNK)
            def _():
                pltpu.make_async_copy(m_hbm.at[pl.ds(0, CH)], rows[b],
                                      gsems[b]).wait()
                if weighted:
                    pltpu.make_async_copy(ew_hbm.at[pl.ds(0, CH)], ewb[b],
                                          esems[b]).wait()
                    def _scale(g, _):
                        wv = ewb[b][pl.ds(g * 16, 16)]
                        for l in range(16):
                            wl = wv.at[jnp.full((16,), l, jnp.int32)].get(
                                mode="promise_in_bounds")
                            r = g * 16 + l
                            for cc in range(D // 16):
                                sl = pl.ds(cc * 16, 16)
                                rows[b][r, sl] = rows[b][r, sl] * wl
                        return 0
                    lax.fori_loop(0, CH // 16, _scale, 0)
                pltpu.make_async_copy(dst_hbm.at[pl.ds(0, CH)], dstb[b],
                                      dsems[b]).wait()
                pltpu.async_copy(rows[b], acc_sh.at[dstb[b]], ssems[b],
                                 add=True)

            @pl.when((k >= 1) & (k <= NCHUNK))
            def _():
                pltpu.make_async_copy(rows[bprev], acc_sh.at[pl.ds(0, CH)],
                                      ssems[bprev]).wait()

            @pl.when(k + NB - 1 < NCHUNK)
            def _():
                pltpu.async_copy(
                    m_hbm.at[src_v.at[pl.ds((k + NB - 1) * CH, CH)]],
                    rows[bprev], gsems[bprev])
                pltpu.async_copy(
                    dst_hbm.at[pl.ds(wid * EPW + (k + NB - 1) * CH, CH)],
                    dstb[bprev], dsems[bprev])
                if weighted:
                    pltpu.async_copy(
                        ew_hbm.at[pl.ds(wid * EPW + (k + NB - 1) * CH, CH)],
                        ewb[bprev], esems[bprev])
        return 0
    lax.fori_loop(0, NOUT, _outer, 0)
    plsc.subcore_barrier()

    for j in range(ROWIT):
        cid = j * NS + s
        @pl.when(cid < NROWCH)
        def _():
            pltpu.sync_copy(acc_sh.at[pl.ds(cid * CH, CH)], rows[j % NB])
            pltpu.sync_copy(rows[j % NB], out_hbm.at[c].at[pl.ds(cid * CH, CH)])


def _sc_agg(m, src, dst, ew):
    weighted = ew is not None
    mesh = plsc.VectorSubcoreMesh(core_axis_name="c", subcore_axis_name="s",
                                  num_cores=NC, num_subcores=NS)
    scratch = [
        pltpu.VMEM((EPW,), jnp.int32),
        [pltpu.VMEM((CH, D), jnp.float32) for _ in range(NB)],
        [pltpu.VMEM((CH,), jnp.int32) for _ in range(NB)],
    ]
    if weighted:
        scratch.append([pltpu.VMEM((CH,), jnp.float32) for _ in range(NB)])
    scratch += [
        pltpu.VMEM_SHARED((N, D), jnp.float32),
        [pltpu.SemaphoreType.DMA for _ in range(NB)],
        [pltpu.SemaphoreType.DMA for _ in range(NB)],
        [pltpu.SemaphoreType.DMA for _ in range(NB)],
    ]
    if weighted:
        scratch.append([pltpu.SemaphoreType.DMA for _ in range(NB)])
    fn = pl.kernel(
        functools.partial(_sc_agg_body, weighted),
        out_type=jax.ShapeDtypeStruct((NC, N, D), jnp.float32),
        mesh=mesh,
        scratch_types=scratch,
    )
    if weighted:
        return fn(m, src, dst, ew)
    return fn(m, src, dst)


def _conv(o, W, wiT, whT, bi, bh, src, dst, ew, leaky_last):
    num_layers = W.shape[0]
    for i in range(num_layers):
        m = _mm(o, W[i])
        parts = _sc_agg(m, src, dst, ew)
        o = _gru(parts[0], parts[1], o, wiT, whT, bi, bh,
                 leaky=(leaky_last and i == num_layers - 1))
    return o


def kernel(x, edge_index, edge_weights, batch,
           c1W, c1wi, c1wh, c1bi, c1bh,
           c2W, c2wi, c2wh, c2bi, c2bh,
           c3W, c3wi, c3wh, c3bi, c3bh,
           c4W, c4wi, c4wh, c4bi, c4bh,
           c5W, c5wi, c5wh, c5bi, c5bh,
           g1w, g1b, g1m, g2w, g2b, g2m,
           g3w, g3b, g3m, g4w, g4b, g4m):
    src = edge_index[0]
    dst = edge_index[1]
    ew = edge_weights
    batch3d = batch.reshape(NBLK, 1, BM)

    def prep(wi, wh, bi, bh):
        return wi.T, wh.T, bi.reshape(1, 3 * D), bh.reshape(1, 3 * D)

    def gprep(w, b, ms):
        return w.reshape(1, D), b.reshape(1, D), ms.reshape(1, D)

    def normed(o, gw, gb, gm, leaky_after):
        wr, br, msr = gprep(gw, gb, gm)
        s1 = _stats1(o)
        s2 = _stats2(o, s1, msr)
        return _norm(o, s1, s2, wr, br, msr, leaky_after=leaky_after)

    o = _conv(x, c1W, *prep(c1wi, c1wh, c1bi, c1bh), src, dst, ew,
              leaky_last=False)
    o = normed(o, g1w, g1b, g1m, leaky_after=True)

    o = _conv(o, c2W, *prep(c2wi, c2wh, c2bi, c2bh), src, dst, ew,
              leaky_last=True)
    o = normed(o, g2w, g2b, g2m, leaky_after=False)

    o = _conv(o, c3W, *prep(c3wi, c3wh, c3bi, c3bh), src, dst, ew,
              leaky_last=True)
    o = normed(o, g3w, g3b, g3m, leaky_after=False)

    o = _conv(o, c4W, *prep(c4wi, c4wh, c4bi, c4bh), src, dst, ew,
              leaky_last=True)
    o = normed(o, g4w, g4b, g4m, leaky_after=False)

    o = _conv(o, c5W, *prep(c5wi, c5wh, c5bi, c5bh), src, dst, None,
              leaky_last=True)

    return _pool(batch3d, o)


# speedup vs baseline: 1.1316x; 1.0249x over previous
import functools

import jax
import jax.numpy as jnp
from jax import lax
from jax.experimental import pallas as pl
from jax.experimental.pallas import tpu as pltpu
from jax.experimental.pallas import tpu_sc as plsc

N = 10000
E = 320000
G = 64
D = 128

BM = 1000
NBLK = N // BM

NC = 2
NS = 16
NW = NC * NS
EPW = E // NW
CH = 80
NCHUNK = EPW // CH
NROWCH = N // CH
ROWIT = (NROWCH + NS - 1) // NS
NB = 3
NOUT = (NCHUNK + NB) // NB


def _mm_body(x_ref, w_ref, o_ref):
    o_ref[...] = jnp.dot(x_ref[...], w_ref[...],
                         preferred_element_type=jnp.float32)


def _mm(x, w):
    return pl.pallas_call(
        _mm_body,
        grid=(NBLK,),
        in_specs=[pl.BlockSpec((BM, D), lambda i: (i, 0)),
                  pl.BlockSpec((D, D), lambda i: (0, 0))],
        out_specs=pl.BlockSpec((BM, D), lambda i: (i, 0)),
        out_shape=jax.ShapeDtypeStruct((N, D), jnp.float32),
    )(x, w)


def _gru_body(a_ref, b_ref, h_ref, wiT_ref, whT_ref, bi_ref, bh_ref, *refs,
              leaky, stats):
    if stats:
        o_ref, s1_ref = refs
    else:
        (o_ref,) = refs
    agg = a_ref[...] + b_ref[...]
    h = h_ref[...]
    gi = jnp.dot(agg, wiT_ref[...], preferred_element_type=jnp.float32) \
        + bi_ref[...]
    gh = jnp.dot(h, whT_ref[...], preferred_element_type=jnp.float32) \
        + bh_ref[...]
    i_r, i_z, i_n = gi[:, 0:D], gi[:, D:2 * D], gi[:, 2 * D:3 * D]
    h_r, h_z, h_n = gh[:, 0:D], gh[:, D:2 * D], gh[:, 2 * D:3 * D]
    r = jax.nn.sigmoid(i_r + h_r)
    z = jax.nn.sigmoid(i_z + h_z)
    n = jnp.tanh(i_n + r * h_n)
    o = (1.0 - z) * n + z * h
    if leaky:
        o = jnp.where(o >= 0, o, 0.01 * o)
    o_ref[...] = o
    if stats:
        i = pl.program_id(0)
        ps1 = jnp.sum(o, axis=0, keepdims=True)

        @pl.when(i == 0)
        def _():
            s1_ref[...] = ps1

        @pl.when(i > 0)
        def _():
            s1_ref[...] = s1_ref[...] + ps1


def _gru(a, b, h, wiT, whT, bi, bh, leaky, stats=False):
    out_specs = [pl.BlockSpec((BM, D), lambda i: (i, 0))]
    out_shape = [jax.ShapeDtypeStruct((N, D), jnp.float32)]
    if stats:
        out_specs.append(pl.BlockSpec((1, D), lambda i: (0, 0)))
        out_shape.append(jax.ShapeDtypeStruct((1, D), jnp.float32))
    return pl.pallas_call(
        functools.partial(_gru_body, leaky=leaky, stats=stats),
        grid=(NBLK,),
        in_specs=[pl.BlockSpec((BM, D), lambda i: (i, 0)),
                  pl.BlockSpec((BM, D), lambda i: (i, 0)),
                  pl.BlockSpec((BM, D), lambda i: (i, 0)),
                  pl.BlockSpec((D, 3 * D), lambda i: (0, 0)),
                  pl.BlockSpec((D, 3 * D), lambda i: (0, 0)),
                  pl.BlockSpec((1, 3 * D), lambda i: (0, 0)),
                  pl.BlockSpec((1, 3 * D), lambda i: (0, 0))],
        out_specs=out_specs,
        out_shape=out_shape,
    )(a, b, h, wiT, whT, bi, bh)


def _stats1_body(x_ref, s1_ref):
    i = pl.program_id(0)
    ps1 = jnp.sum(x_ref[...], axis=0, keepdims=True)

    @pl.when(i == 0)
    def _():
        s1_ref[...] = ps1

    @pl.when(i > 0)
    def _():
        s1_ref[...] = s1_ref[...] + ps1


def _stats1(x):
    return pl.pallas_call(
        _stats1_body,
        grid=(NBLK,),
        in_specs=[pl.BlockSpec((BM, D), lambda i: (i, 0))],
        out_specs=pl.BlockSpec((1, D), lambda i: (0, 0)),
        out_shape=jax.ShapeDtypeStruct((1, D), jnp.float32),
    )(x)


def _stats2_body(x_ref, s1_ref, ms_ref, s2_ref):
    i = pl.program_id(0)
    cm = s1_ref[...] * (1.0 / N) * ms_ref[...]
    d = x_ref[...] - cm
    ps2 = jnp.sum(d * d, axis=0, keepdims=True)

    @pl.when(i == 0)
    def _():
        s2_ref[...] = ps2

    @pl.when(i > 0)
    def _():
        s2_ref[...] = s2_ref[...] + ps2


def _stats2(x, s1, ms):
    return pl.pallas_call(
        _stats2_body,
        grid=(NBLK,),
        in_specs=[pl.BlockSpec((BM, D), lambda i: (i, 0)),
                  pl.BlockSpec((1, D), lambda i: (0, 0)),
                  pl.BlockSpec((1, D), lambda i: (0, 0))],
        out_specs=pl.BlockSpec((1, D), lambda i: (0, 0)),
        out_shape=jax.ShapeDtypeStruct((1, D), jnp.float32),
    )(x, s1, ms)


def _norm_body(x_ref, s1_ref, s2_ref, w_ref, b_ref, ms_ref, o_ref, *,
               leaky_after):
    cm = s1_ref[...] * (1.0 / N) * ms_ref[...]
    var = s2_ref[...] * (1.0 / N)
    rstd = jax.lax.rsqrt(var + 1e-5)
    y = w_ref[...] * (x_ref[...] - cm) * rstd + b_ref[...]
    if leaky_after:
        y = jnp.where(y >= 0, y, 0.01 * y)
    o_ref[...] = y


def _norm_mm_body(x_ref, s1_ref, s2_ref, w_ref, b_ref, ms_ref, wmat_ref,
                  o_ref, m_ref, *, leaky_after):
    cm = s1_ref[...] * (1.0 / N) * ms_ref[...]
    var = s2_ref[...] * (1.0 / N)
    rstd = jax.lax.rsqrt(var + 1e-5)
    y = w_ref[...] * (x_ref[...] - cm) * rstd + b_ref[...]
    if leaky_after:
        y = jnp.where(y >= 0, y, 0.01 * y)
    o_ref[...] = y
    m_ref[...] = jnp.dot(y, wmat_ref[...], preferred_element_type=jnp.float32)


def _norm_mm(x, s1, s2, w, b, ms, wmat, leaky_after):
    return pl.pallas_call(
        functools.partial(_norm_mm_body, leaky_after=leaky_after),
        grid=(NBLK,),
        in_specs=[pl.BlockSpec((BM, D), lambda i: (i, 0)),
                  pl.BlockSpec((1, D), lambda i: (0, 0)),
                  pl.BlockSpec((1, D), lambda i: (0, 0)),
                  pl.BlockSpec((1, D), lambda i: (0, 0)),
                  pl.BlockSpec((1, D), lambda i: (0, 0)),
                  pl.BlockSpec((1, D), lambda i: (0, 0)),
                  pl.BlockSpec((D, D), lambda i: (0, 0))],
        out_specs=[pl.BlockSpec((BM, D), lambda i: (i, 0)),
                   pl.BlockSpec((BM, D), lambda i: (i, 0))],
        out_shape=[jax.ShapeDtypeStruct((N, D), jnp.float32),
                   jax.ShapeDtypeStruct((N, D), jnp.float32)],
    )(x, s1, s2, w, b, ms, wmat)


def _norm(x, s1, s2, w, b, ms, leaky_after):
    return pl.pallas_call(
        functools.partial(_norm_body, leaky_after=leaky_after),
        grid=(NBLK,),
        in_specs=[pl.BlockSpec((BM, D), lambda i: (i, 0)),
                  pl.BlockSpec((1, D), lambda i: (0, 0)),
                  pl.BlockSpec((1, D), lambda i: (0, 0)),
                  pl.BlockSpec((1, D), lambda i: (0, 0)),
                  pl.BlockSpec((1, D), lambda i: (0, 0)),
                  pl.BlockSpec((1, D), lambda i: (0, 0))],
        out_specs=pl.BlockSpec((BM, D), lambda i: (i, 0)),
        out_shape=jax.ShapeDtypeStruct((N, D), jnp.float32),
    )(x, s1, s2, w, b, ms)


def _pool_body(b_ref, x_ref, o_ref):
    i = pl.program_id(0)
    b = b_ref[0, 0, :]
    rows = lax.broadcasted_iota(jnp.int32, (G, BM), 0)
    oh = (rows == b[None, :]).astype(jnp.float32)
    acc = jnp.dot(oh, x_ref[...], preferred_element_type=jnp.float32)

    @pl.when(i == 0)
    def _():
        o_ref[...] = acc

    @pl.when(i > 0)
    def _():
        o_ref[...] = o_ref[...] + acc


def _pool(batch3d, x):
    return pl.pallas_call(
        _pool_body,
        grid=(NBLK,),
        in_specs=[pl.BlockSpec((1, 1, BM), lambda i: (i, 0, 0)),
                  pl.BlockSpec((BM, D), lambda i: (i, 0))],
        out_specs=pl.BlockSpec((G, D), lambda i: (0, 0)),
        out_shape=jax.ShapeDtypeStruct((G, D), jnp.float32),
    )(batch3d, x)


def _sc_agg_body(weighted, *refs):
    if weighted:
        (m_hbm, src_hbm, dst_hbm, ew_hbm, out_hbm,
         src_v, rows, dstb, ewb, acc_sh, gsems, ssems, dsems, esems) = refs
    else:
        (m_hbm, src_hbm, dst_hbm, out_hbm,
         src_v, rows, dstb, acc_sh, gsems, ssems, dsems) = refs
        ew_hbm = ewb = esems = None
    c = lax.axis_index("c")
    s = lax.axis_index("s")
    wid = s * NC + c

    pltpu.sync_copy(src_hbm.at[pl.ds(wid * EPW, EPW)], src_v)

    def _zrow(i, _):
        for cc in range(D // 16):
            rows[0][i, pl.ds(cc * 16, 16)] = jnp.zeros((16,), jnp.float32)
        return 0
    lax.fori_loop(0, CH, _zrow, 0)
    for j in range(ROWIT):
        cid = j * NS + s
        @pl.when(cid < NROWCH)
        def _():
            pltpu.sync_copy(rows[0], acc_sh.at[pl.ds(cid * CH, CH)])
    plsc.subcore_barrier()

    for j in range(NB - 1):
        pltpu.async_copy(m_hbm.at[src_v.at[pl.ds(j * CH, CH)]], rows[j],
                         gsems[j])
        pltpu.async_copy(dst_hbm.at[pl.ds(wid * EPW + j * CH, CH)], dstb[j],
                         dsems[j])
        if weighted:
            pltpu.async_copy(ew_hbm.at[pl.ds(wid * EPW + j * CH, CH)],
                             ewb[j], esems[j])

    def _outer(k2, _):
        for b in range(NB):
            k = k2 * NB + b
            bprev = (b + NB - 1) % NB

            @pl.when(k < NCHUNK)
            def _():
                pltpu.make_async_copy(m_hbm.at[pl.ds(0, CH)], rows[b],
                                      gsems[b]).wait()
                if weighted:
                    pltpu.make_async_copy(ew_hbm.at[pl.ds(0, CH)], ewb[b],
                                          esems[b]).wait()
                    def _scale(g, _):
                        wv = ewb[b][pl.ds(g * 16, 16)]
                        for l in range(16):
                            wl = wv.at[jnp.full((16,), l, jnp.int32)].get(
                                mode="promise_in_bounds")
                            r = g * 16 + l
                            for cc in range(D // 16):
                                sl = pl.ds(cc * 16, 16)
                                rows[b][r, sl] = rows[b][r, sl] * wl
                        return 0
                    lax.fori_loop(0, CH // 16, _scale, 0)
                pltpu.make_async_copy(dst_hbm.at[pl.ds(0, CH)], dstb[b],
                                      dsems[b]).wait()
                pltpu.async_copy(rows[b], acc_sh.at[dstb[b]], ssems[b],
                                 add=True)

            @pl.when((k >= 1) & (k <= NCHUNK))
            def _():
                pltpu.make_async_copy(rows[bprev], acc_sh.at[pl.ds(0, CH)],
                                      ssems[bprev]).wait()

            @pl.when(k + NB - 1 < NCHUNK)
            def _():
                pltpu.async_copy(
                    m_hbm.at[src_v.at[pl.ds((k + NB - 1) * CH, CH)]],
                    rows[bprev], gsems[bprev])
                pltpu.async_copy(
                    dst_hbm.at[pl.ds(wid * EPW + (k + NB - 1) * CH, CH)],
                    dstb[bprev], dsems[bprev])
                if weighted:
                    pltpu.async_copy(
                        ew_hbm.at[pl.ds(wid * EPW + (k + NB - 1) * CH, CH)],
                        ewb[bprev], esems[bprev])
        return 0
    lax.fori_loop(0, NOUT, _outer, 0)
    plsc.subcore_barrier()

    for j in range(ROWIT):
        cid = j * NS + s
        @pl.when(cid < NROWCH)
        def _():
            pltpu.sync_copy(acc_sh.at[pl.ds(cid * CH, CH)], rows[j % NB])
            pltpu.sync_copy(rows[j % NB], out_hbm.at[c].at[pl.ds(cid * CH, CH)])


def _sc_agg(m, src, dst, ew):
    weighted = ew is not None
    mesh = plsc.VectorSubcoreMesh(core_axis_name="c", subcore_axis_name="s",
                                  num_cores=NC, num_subcores=NS)
    scratch = [
        pltpu.VMEM((EPW,), jnp.int32),
        [pltpu.VMEM((CH, D), jnp.float32) for _ in range(NB)],
        [pltpu.VMEM((CH,), jnp.int32) for _ in range(NB)],
    ]
    if weighted:
        scratch.append([pltpu.VMEM((CH,), jnp.float32) for _ in range(NB)])
    scratch += [
        pltpu.VMEM_SHARED((N, D), jnp.float32),
        [pltpu.SemaphoreType.DMA for _ in range(NB)],
        [pltpu.SemaphoreType.DMA for _ in range(NB)],
        [pltpu.SemaphoreType.DMA for _ in range(NB)],
    ]
    if weighted:
        scratch.append([pltpu.SemaphoreType.DMA for _ in range(NB)])
    fn = pl.kernel(
        functools.partial(_sc_agg_body, weighted),
        out_type=jax.ShapeDtypeStruct((NC, N, D), jnp.float32),
        mesh=mesh,
        scratch_types=scratch,
    )
    if weighted:
        return fn(m, src, dst, ew)
    return fn(m, src, dst)


def _conv(o, W, wiT, whT, bi, bh, src, dst, ew, leaky_last,
          m0=None, stats_last=False):
    num_layers = W.shape[0]
    for i in range(num_layers):
        if i == 0 and m0 is not None:
            m = m0
        else:
            m = _mm(o, W[i])
        parts = _sc_agg(m, src, dst, ew)
        last = (i == num_layers - 1)
        res = _gru(parts[0], parts[1], o, wiT, whT, bi, bh,
                   leaky=(leaky_last and last), stats=(stats_last and last))
        if stats_last and last:
            o, s1 = res
        else:
            (o,) = res
    if stats_last:
        return o, s1
    return o


def kernel(x, edge_index, edge_weights, batch,
           c1W, c1wi, c1wh, c1bi, c1bh,
           c2W, c2wi, c2wh, c2bi, c2bh,
           c3W, c3wi, c3wh, c3bi, c3bh,
           c4W, c4wi, c4wh, c4bi, c4bh,
           c5W, c5wi, c5wh, c5bi, c5bh,
           g1w, g1b, g1m, g2w, g2b, g2m,
           g3w, g3b, g3m, g4w, g4b, g4m):
    src = edge_index[0]
    dst = edge_index[1]
    ew = edge_weights
    batch3d = batch.reshape(NBLK, 1, BM)

    def prep(wi, wh, bi, bh):
        return wi.T, wh.T, bi.reshape(1, 3 * D), bh.reshape(1, 3 * D)

    def gprep(w, b, ms):
        return w.reshape(1, D), b.reshape(1, D), ms.reshape(1, D)

    def normed_mm(o, s1, gw, gb, gm, wmat, leaky_after):
        wr, br, msr = gprep(gw, gb, gm)
        s2 = _stats2(o, s1, msr)
        return _norm_mm(o, s1, s2, wr, br, msr, wmat, leaky_after=leaky_after)

    o, s1 = _conv(x, c1W, *prep(c1wi, c1wh, c1bi, c1bh), src, dst, ew,
                  leaky_last=False, stats_last=True)
    o, m0 = normed_mm(o, s1, g1w, g1b, g1m, c2W[0], leaky_after=True)

    o, s1 = _conv(o, c2W, *prep(c2wi, c2wh, c2bi, c2bh), src, dst, ew,
                  leaky_last=True, m0=m0, stats_last=True)
    o, m0 = normed_mm(o, s1, g2w, g2b, g2m, c3W[0], leaky_after=False)

    o, s1 = _conv(o, c3W, *prep(c3wi, c3wh, c3bi, c3bh), src, dst, ew,
                  leaky_last=True, m0=m0, stats_last=True)
    o, m0 = normed_mm(o, s1, g3w, g3b, g3m, c4W[0], leaky_after=False)

    o, s1 = _conv(o, c4W, *prep(c4wi, c4wh, c4bi, c4bh), src, dst, ew,
                  leaky_last=True, m0=m0, stats_last=True)
    o, m0 = normed_mm(o, s1, g4w, g4b, g4m, c5W[0], leaky_after=False)

    o = _conv(o, c5W, *prep(c5wi, c5wh, c5bi, c5bh), src, dst, None,
              leaky_last=True, m0=m0)

    return _pool(batch3d, o)
